# Initial kernel scaffold; baseline (speedup 1.0000x reference)
import jax
import jax.numpy as jnp
from jax.experimental import pallas as pl


def kernel(x, edge_index, edge_attr, batch, Wnp, bnp, Wep, bep, W1, as1, ad1, b1, We1, ae1, W2, as2, ad2, b2, We2, ae2, Wp1, asp1, adp1, bp1, wproj1, bproj1, Wp2, asp2, adp2, bp2, wproj2, bproj2):
    raise NotImplementedError("write your pallas kernel here")


# speedup vs baseline: 7.0592x; 7.0592x over previous
import functools

import jax
import jax.numpy as jnp
from jax import lax
from jax.experimental import pallas as pl
from jax.experimental.pallas import tpu as pltpu
from jax.experimental.pallas import tpu_sc as plsc

F32 = jnp.float32
I32 = jnp.int32

N = 10000
E = 160000
D = 512
C = 8
CW = 64
G = 16
NC = 2
NS = 16
L = 16
NPAD = 10240
EP = E // NS
GE = 80
NG = EP // GE
EPS = E // (NC * NS)
SP = N // NS
RED = NPAD // NS
BLKN = 1000
BLKE = 2000


def _leaky(a):
    return jnp.where(a > 0, a, 0.2 * a)


def _tc_lin0_body(x_ref, wnp_ref, bnp_ref, w1_ref, a1_ref, hc_ref, asad_ref):
    h0 = jnp.dot(x_ref[...], wnp_ref[...], preferred_element_type=F32) + bnp_ref[...]
    h1 = jnp.dot(h0, w1_ref[...], preferred_element_type=F32)
    for c in range(C):
        hc_ref[c] = h1[:, c * CW:(c + 1) * CW]
    asad_ref[...] = jnp.dot(h1, a1_ref[...], preferred_element_type=F32)


def _tc_lin0(x, wnp, bnp, w1, a1):
    nb = N // BLKN
    return pl.pallas_call(
        _tc_lin0_body,
        grid=(nb,),
        in_specs=[
            pl.BlockSpec((BLKN, 256), lambda i: (i, 0)),
            pl.BlockSpec((256, 256), lambda i: (0, 0)),
            pl.BlockSpec((1, 256), lambda i: (0, 0)),
            pl.BlockSpec((256, D), lambda i: (0, 0)),
            pl.BlockSpec((D, 8), lambda i: (0, 0)),
        ],
        out_specs=[
            pl.BlockSpec((C, BLKN, CW), lambda i: (0, i, 0)),
            pl.BlockSpec((BLKN, 8), lambda i: (i, 0)),
        ],
        out_shape=[
            jax.ShapeDtypeStruct((C, N, CW), F32),
            jax.ShapeDtypeStruct((N, 8), F32),
        ],
    )(x, wnp, bnp, w1, a1)


def _tc_es_body(ea_ref, u8_ref, c8_ref, es_ref):
    es_ref[...] = jnp.dot(ea_ref[...], u8_ref[...], preferred_element_type=F32) + c8_ref[...]


def _tc_es(ea, u8, c8):
    nb = E // BLKE
    return pl.pallas_call(
        _tc_es_body,
        grid=(nb,),
        in_specs=[
            pl.BlockSpec((BLKE, 256), lambda i: (i, 0)),
            pl.BlockSpec((256, 8), lambda i: (0, 0)),
            pl.BlockSpec((1, 8), lambda i: (0, 0)),
        ],
        out_specs=pl.BlockSpec((BLKE, 8), lambda i: (i, 0)),
        out_shape=jax.ShapeDtypeStruct((E, 8), F32),
    )(ea, u8, c8)


def _combine_block(outp_ref, hcin_ref, asad_ref, den_ref, b_ref, hsc, exl_extra, relu):
    a_s = asad_ref[:, 0:1]
    a_d = asad_ref[:, 1:2]
    exl = jnp.exp(_leaky(a_s + a_d + exl_extra))
    dt = den_ref[...] + exl + 1e-16
    for c in range(C):
        ob = (outp_ref[c] + exl * hcin_ref[c]) / dt + b_ref[:, c * CW:(c + 1) * CW]
        if relu:
            ob = jnp.maximum(ob, 0.0)
        hsc[:, c * CW:(c + 1) * CW] = ob


def _tc_comb1_body(outp_ref, hcin_ref, asad_ref, ssum_ref, ideg_ref, den_ref,
                   b_ref, w2_ref, a2_ref, hc2_ref, asad2_ref, hsc):
    m = ssum_ref[...] / jnp.maximum(ideg_ref[...], 1.0)
    _combine_block(outp_ref, hcin_ref, asad_ref, den_ref, b_ref, hsc, m, True)
    h = hsc[...]
    h2 = jnp.dot(h, w2_ref[...], preferred_element_type=F32)
    for c in range(C):
        hc2_ref[c] = h2[:, c * CW:(c + 1) * CW]
    asad2_ref[...] = jnp.dot(h2, a2_ref[...], preferred_element_type=F32)


def _tc_comb1(outp, hcin, asad, ssum, ideg, den, b, w2, a2):
    nb = N // BLKN
    col = pl.BlockSpec((BLKN, 1), lambda i: (i, 0))
    return pl.pallas_call(
        _tc_comb1_body,
        grid=(nb,),
        in_specs=[
            pl.BlockSpec((C, BLKN, CW), lambda i: (0, i, 0)),
            pl.BlockSpec((C, BLKN, CW), lambda i: (0, i, 0)),
            pl.BlockSpec((BLKN, 8), lambda i: (i, 0)),
            col, col, col,
            pl.BlockSpec((1, D), lambda i: (0, 0)),
            pl.BlockSpec((D, D), lambda i: (0, 0)),
            pl.BlockSpec((D, 8), lambda i: (0, 0)),
        ],
        out_specs=[
            pl.BlockSpec((C, BLKN, CW), lambda i: (0, i, 0)),
            pl.BlockSpec((BLKN, 8), lambda i: (i, 0)),
        ],
        out_shape=[
            jax.ShapeDtypeStruct((C, N, CW), F32),
            jax.ShapeDtypeStruct((N, 8), F32),
        ],
        scratch_shapes=[pltpu.VMEM((BLKN, D), F32)],
    )(outp, hcin, asad, ssum, ideg, den, b, w2, a2)


def _tc_comb2_body(outp_ref, hcin_ref, asad_ref, ssum_ref, ideg_ref, den_ref,
                   b_ref, wp1_ref, ap1_ref, wp2_ref, ap2_ref,
                   hp1_ref, asadp1_ref, hp2_ref, asadp2_ref, hsc):
    m = ssum_ref[...] / jnp.maximum(ideg_ref[...], 1.0)
    _combine_block(outp_ref, hcin_ref, asad_ref, den_ref, b_ref, hsc, m, True)
    h = hsc[...]
    hp1 = jnp.dot(h, wp1_ref[...], preferred_element_type=F32)
    for c in range(C):
        hp1_ref[c] = hp1[:, c * CW:(c + 1) * CW]
    asadp1_ref[...] = jnp.dot(hp1, ap1_ref[...], preferred_element_type=F32)
    hp2 = jnp.dot(h, wp2_ref[...], preferred_element_type=F32)
    for c in range(C):
        hp2_ref[c] = hp2[:, c * CW:(c + 1) * CW]
    asadp2_ref[...] = jnp.dot(hp2, ap2_ref[...], preferred_element_type=F32)


def _tc_comb2(outp, hcin, asad, ssum, ideg, den, b, wp1, ap1, wp2, ap2):
    nb = N // BLKN
    col = pl.BlockSpec((BLKN, 1), lambda i: (i, 0))
    return pl.pallas_call(
        _tc_comb2_body,
        grid=(nb,),
        in_specs=[
            pl.BlockSpec((C, BLKN, CW), lambda i: (0, i, 0)),
            pl.BlockSpec((C, BLKN, CW), lambda i: (0, i, 0)),
            pl.BlockSpec((BLKN, 8), lambda i: (i, 0)),
            col, col, col,
            pl.BlockSpec((1, D), lambda i: (0, 0)),
            pl.BlockSpec((D, D), lambda i: (0, 0)),
            pl.BlockSpec((D, 8), lambda i: (0, 0)),
            pl.BlockSpec((D, D), lambda i: (0, 0)),
            pl.BlockSpec((D, 8), lambda i: (0, 0)),
        ],
        out_specs=[
            pl.BlockSpec((C, BLKN, CW), lambda i: (0, i, 0)),
            pl.BlockSpec((BLKN, 8), lambda i: (i, 0)),
            pl.BlockSpec((C, BLKN, CW), lambda i: (0, i, 0)),
            pl.BlockSpec((BLKN, 8), lambda i: (i, 0)),
        ],
        out_shape=[
            jax.ShapeDtypeStruct((C, N, CW), F32),
            jax.ShapeDtypeStruct((N, 8), F32),
            jax.ShapeDtypeStruct((C, N, CW), F32),
            jax.ShapeDtypeStruct((N, 8), F32),
        ],
        scratch_shapes=[pltpu.VMEM((BLKN, D), F32)],
    )(outp, hcin, asad, ssum, ideg, den, b, wp1, ap1, wp2, ap2)


def _tc_final_body(outp1_ref, h1c_ref, as1_ref, den1_ref, b1_ref, wj1_ref, bj1_ref,
                   outp2_ref, h2c_ref, as2_ref, den2_ref, b2_ref, wj2_ref, bj2_ref,
                   batch_ref, out_ref, psc, g1sc, g2sc, cntsc):
    i = pl.program_id(0)

    @pl.when(i == 0)
    def _():
        g1sc[...] = jnp.zeros_like(g1sc)
        g2sc[...] = jnp.zeros_like(g2sc)
        cntsc[...] = jnp.zeros_like(cntsc)

    bat = batch_ref[...]
    oh = (lax.broadcasted_iota(I32, (BLKN, G), 1) == bat).astype(F32)
    ones = jnp.ones((BLKN, 1), F32)
    cntsc[:, 0:1] += lax.dot_general(oh, ones, (((0,), (0,)), ((), ())),
                                     preferred_element_type=F32)

    pools = [
        (outp1_ref, h1c_ref, as1_ref, den1_ref, b1_ref, wj1_ref, bj1_ref, g1sc),
        (outp2_ref, h2c_ref, as2_ref, den2_ref, b2_ref, wj2_ref, bj2_ref, g2sc),
    ]
    for (outp_ref, hc_ref, as_ref, den_ref, b_ref, wj_ref, bj_ref, gsc) in pools:
        _combine_block(outp_ref, hc_ref, as_ref, den_ref, b_ref, psc, 0.0, False)
        p = psc[...]
        sv = jnp.dot(p, wj_ref[...], preferred_element_type=F32)
        score = jax.nn.sigmoid(sv[:, 0:1] + bj_ref[:, 0:1])
        xw = score * p
        gsc[...] += lax.dot_general(oh, xw, (((0,), (0,)), ((), ())),
                                    preferred_element_type=F32)

    @pl.when(i == pl.num_programs(0) - 1)
    def _():
        cnt = jnp.maximum(cntsc[:, 0:1], 1.0)
        out_ref[:, :D] = g1sc[...] / cnt
        out_ref[:, D:] = g2sc[...] / cnt


def _tc_final(outp1, h1c, as1, den1, b1, wj1, bj1,
              outp2, h2c, as2, den2, b2, wj2, bj2, batchr):
    nb = N // BLKN
    col = pl.BlockSpec((BLKN, 1), lambda i: (i, 0))
    chunks = pl.BlockSpec((C, BLKN, CW), lambda i: (0, i, 0))
    asads = pl.BlockSpec((BLKN, 8), lambda i: (i, 0))
    full_b = pl.BlockSpec((1, D), lambda i: (0, 0))
    wj_s = pl.BlockSpec((D, 8), lambda i: (0, 0))
    bj_s = pl.BlockSpec((1, 8), lambda i: (0, 0))
    return pl.pallas_call(
        _tc_final_body,
        grid=(nb,),
        in_specs=[
            chunks, chunks, asads, col, full_b, wj_s, bj_s,
            chunks, chunks, asads, col, full_b, wj_s, bj_s,
            pl.BlockSpec((BLKN, 1), lambda i: (i, 0)),
        ],
        out_specs=pl.BlockSpec((G, 2 * D), lambda i: (0, 0)),
        out_shape=jax.ShapeDtypeStruct((G, 2 * D), F32),
        scratch_shapes=[
            pltpu.VMEM((BLKN, D), F32),
            pltpu.VMEM((G, D), F32),
            pltpu.VMEM((G, D), F32),
            pltpu.VMEM((G, CW), F32),
        ],
    )(outp1, h1c, as1, den1, b1, wj1, bj1,
      outp2, h2c, as2, den2, b2, wj2, bj2, batchr)


_MESH = dict(core_axis_name="c", subcore_axis_name="s", num_cores=NC,
             num_subcores=NS)


def _sc_conv_body(hc, srcf, dst2, sflat, asf, adf, zn,
                  outp, den_out,
                  a_s_tbl, a_d_tbl, srcb, dstb, sb, exb, dtbl,
                  rows, rtmp, acc, spred):
    cid = lax.axis_index("c")
    sid = lax.axis_index("s")
    z16 = jnp.zeros((L,), F32)

    pltpu.sync_copy(asf, a_s_tbl.at[pl.ds(0, N)])
    pltpu.sync_copy(adf, a_d_tbl.at[pl.ds(0, N)])
    e0 = pl.multiple_of(sid * EP, 8)
    pltpu.sync_copy(srcf.at[pl.ds(e0, EP)], srcb)
    pltpu.sync_copy(dst2.at[sid], dstb)
    pltpu.sync_copy(sflat.at[pl.ds(e0, EP)], sb)

    def zd(v, _):
        dtbl[pl.ds(v * L, L)] = z16
        return _
    lax.fori_loop(0, NPAD // L, zd, None)

    for ci in range(C // NC):
        cc = cid * (C // NC) + ci
        r0 = pl.multiple_of(sid * 632, 8)

        @pl.when(sid < NS - 1)
        def _():
            sl = pl.ds(r0, 632)
            pltpu.sync_copy(zn.at[sl], acc.at[sl])

        @pl.when(sid == NS - 1)
        def _():
            sl = pl.ds(9480, 520)
            pltpu.sync_copy(zn.at[sl], acc.at[sl])
        plsc.subcore_barrier()

        hcc = hc.at[cc]

        def group(g, _):
            base = g * GE
            if True:
                pass
            if ci == 0:
                for j in range(GE // L):
                    off = base + j * L
                    si = srcb[pl.ds(off, L)]
                    di = dstb[g, pl.ds(j * L, L)]
                    av = (plsc.load_gather(a_s_tbl, [si])
                          + plsc.load_gather(a_d_tbl, [di])
                          + sb[pl.ds(off, L)])
                    ex = jnp.exp(_leaky(av))
                    exb[pl.ds(off, L)] = ex

                    @pl.when(cid == 0)
                    def _():
                        plsc.addupdate_scatter(dtbl, [di], ex)

            pltpu.sync_copy(hcc.at[srcb.at[pl.ds(base, GE)]], rows)
            for j in range(GE):
                exv = plsc.load_gather(exb, [jnp.full((L,), base + j, I32)])
                for k in range(CW // L):
                    sl = pl.ds(k * L, L)
                    rows[j, sl] = rows[j, sl] * exv
            pltpu.sync_copy(rows, acc.at[dstb.at[g]], add=True)
            return _

        lax.fori_loop(0, NG, group, None)
        plsc.subcore_barrier()

        @pl.when(sid < NS - 1)
        def _():
            sl = pl.ds(r0, 632)
            pltpu.sync_copy(acc.at[sl], outp.at[cc].at[sl])

        @pl.when(sid == NS - 1)
        def _():
            sl = pl.ds(9480, 520)
            pltpu.sync_copy(acc.at[sl], outp.at[cc].at[sl])
        plsc.subcore_barrier()

    @pl.when(cid == 0)
    def _():
        for r in range(NPAD // 2560):
            rb = r * 2560
            s0 = pl.multiple_of(sid * 160, 8)
            n0 = pl.multiple_of(rb + sid * 160, 8)
            pltpu.sync_copy(dtbl.at[pl.ds(rb, 2560)], spred.at[sid])
            plsc.subcore_barrier()

            def zr(v, _):
                dtbl[pl.ds(n0 + v * L, L)] = z16
                return _
            lax.fori_loop(0, 160 // L, zr, None)
            for tt in range(NS):
                pltpu.sync_copy(spred.at[tt].at[pl.ds(s0, 160)],
                                rtmp.at[pl.ds(0, 160)])

                def addv(v, _):
                    sl = pl.ds(n0 + v * L, L)
                    dtbl[sl] = dtbl[sl] + rtmp[pl.ds(v * L, L)]
                    return _
                lax.fori_loop(0, 160 // L, addv, None)
            pltpu.sync_copy(dtbl.at[pl.ds(n0, 160)], den_out.at[pl.ds(n0, 160)])
            plsc.subcore_barrier()


def _sc_conv(hc, srcf, dst2, sflat, asf, adf, zn):
    mesh = plsc.VectorSubcoreMesh(**_MESH)
    fn = pl.kernel(
        _sc_conv_body,
        out_type=[
            jax.ShapeDtypeStruct((C, N, CW), F32),
            jax.ShapeDtypeStruct((NPAD,), F32),
        ],
        mesh=mesh,
        compiler_params=pltpu.CompilerParams(needs_layout_passes=False,
                                             use_tc_tiling_on_sc=False),
        scratch_types=[
            pltpu.VMEM((NPAD,), F32),
            pltpu.VMEM((NPAD,), F32),
            pltpu.VMEM((EP,), I32),
            pltpu.VMEM((NG, GE), I32),
            pltpu.VMEM((EP,), F32),
            pltpu.VMEM((EP,), F32),
            pltpu.VMEM((NPAD,), F32),
            pltpu.VMEM((GE, CW), F32),
            pltpu.VMEM((RED,), F32),
            pltpu.VMEM_SHARED((N, CW), F32),
            pltpu.VMEM_SHARED((NS, 2560), F32),
        ],
    )
    return fn(hc, srcf, dst2, sflat, asf, adf, zn)


def _sc_stats_body(dstf, s1f, s2f, s1o, s2o, cnto,
                   dstb, sb1, sb2, t1, t2, tc, redb, rtmp, spred):
    cid = lax.axis_index("c")
    sid = lax.axis_index("s")
    wid = cid * NS + sid
    e0 = pl.multiple_of(wid * EPS, 8)
    iota = lax.iota(I32, L)
    z16 = jnp.zeros((L,), F32)
    ones = jnp.ones((L,), F32)

    pltpu.sync_copy(dstf.at[pl.ds(e0, EPS)], dstb.at[pl.ds(0, EPS)])
    pltpu.sync_copy(s1f.at[pl.ds(e0, EPS)], sb1.at[pl.ds(0, EPS)])
    pltpu.sync_copy(s2f.at[pl.ds(e0, EPS)], sb2.at[pl.ds(0, EPS)])

    for t in (t1, t2, tc):
        def zd(v, _, _t=t):
            _t[pl.ds(v * L, L)] = z16
            return _
        lax.fori_loop(0, NPAD // L, zd, None)

    def grp(g, _):
        off = g * L
        msk = (off + iota) < EPS
        di = dstb[pl.ds(off, L)]
        plsc.addupdate_scatter(t1, [di], sb1[pl.ds(off, L)], mask=msk)
        plsc.addupdate_scatter(t2, [di], sb2[pl.ds(off, L)], mask=msk)
        plsc.addupdate_scatter(tc, [di], ones, mask=msk)
        return _
    lax.fori_loop(0, (EPS + L - 1) // L, grp, None)

    n0 = pl.multiple_of(sid * RED, 8)
    for (tbl, outr) in ((t1, s1o), (t2, s2o), (tc, cnto)):
        plsc.subcore_barrier()
        pltpu.sync_copy(tbl, spred.at[sid])
        plsc.subcore_barrier()

        def zr(v, _):
            redb[pl.ds(v * L, L)] = z16
            return _
        lax.fori_loop(0, RED // L, zr, None)
        for t in range(NS):
            pltpu.sync_copy(spred.at[t].at[pl.ds(n0, RED)], rtmp)

            def addv(v, _):
                sl = pl.ds(v * L, L)
                redb[sl] = redb[sl] + rtmp[sl]
                return _
            lax.fori_loop(0, RED // L, addv, None)
        pltpu.sync_copy(redb, outr.at[cid].at[pl.ds(n0, RED)])


def _sc_stats(dstf, s1f, s2f):
    mesh = plsc.VectorSubcoreMesh(**_MESH)
    fn = pl.kernel(
        _sc_stats_body,
        out_type=[
            jax.ShapeDtypeStruct((NC, NPAD), F32),
            jax.ShapeDtypeStruct((NC, NPAD), F32),
            jax.ShapeDtypeStruct((NC, NPAD), F32),
        ],
        mesh=mesh,
        compiler_params=pltpu.CompilerParams(needs_layout_passes=False,
                                             use_tc_tiling_on_sc=False),
        scratch_types=[
            pltpu.VMEM((EPS + 16, ), I32),
            pltpu.VMEM((EPS + 16, ), F32),
            pltpu.VMEM((EPS + 16, ), F32),
            pltpu.VMEM((NPAD,), F32),
            pltpu.VMEM((NPAD,), F32),
            pltpu.VMEM((NPAD,), F32),
            pltpu.VMEM((RED,), F32),
            pltpu.VMEM((RED,), F32),
            pltpu.VMEM_SHARED((NS, NPAD), F32),
        ],
    )
    return fn(dstf, s1f, s2f)


def kernel(x, edge_index, edge_attr, batch, Wnp, bnp, Wep, bep, W1, as1, ad1, b1,
           We1, ae1, W2, as2, ad2, b2, We2, ae2, Wp1, asp1, adp1, bp1, wproj1,
           bproj1, Wp2, asp2, adp2, bp2, wproj2, bproj2):
    src = edge_index[0]
    dst = edge_index[1]
    dst2 = dst.reshape(NS, NG, GE)

    v1 = We1 @ ae1
    u1 = Wep @ v1
    c1 = bep @ v1
    v2 = We2 @ ae2
    u2 = Wep @ v2
    c2 = bep @ v2
    u8 = jnp.zeros((256, 8), F32).at[:, 0].set(u1).at[:, 1].set(u2)
    c8 = jnp.zeros((1, 8), F32).at[0, 0].set(c1).at[0, 1].set(c2)

    def cols8(a, b):
        return jnp.zeros((D, 8), F32).at[:, 0].set(a).at[:, 1].set(b)

    a1 = cols8(as1, ad1)
    a2 = cols8(as2, ad2)
    ap1 = cols8(asp1, adp1)
    ap2 = cols8(asp2, adp2)
    wj1 = jnp.zeros((D, 8), F32).at[:, 0].set(wproj1[:, 0])
    wj2 = jnp.zeros((D, 8), F32).at[:, 0].set(wproj2[:, 0])
    bj1 = jnp.zeros((1, 8), F32).at[0, 0].set(bproj1[0])
    bj2 = jnp.zeros((1, 8), F32).at[0, 0].set(bproj2[0])
    zn = jnp.zeros((N, CW), F32)
    batchr = batch.reshape(N, 1)

    h1c, asad1 = _tc_lin0(x, Wnp, bnp.reshape(1, 256), W1, a1)
    es = _tc_es(edge_attr, u8, c8)

    s1f = es[:, 0]
    s2f = es[:, 1]
    zedge = jnp.zeros((E,), F32)
    s1p, s2p, cntp = _sc_stats(dst, s1f, s2f)
    ssum1 = (s1p[0] + s1p[1])[:N].reshape(N, 1)
    ssum2 = (s2p[0] + s2p[1])[:N].reshape(N, 1)
    ideg = (cntp[0] + cntp[1])[:N].reshape(N, 1)

    outp1, den1 = _sc_conv(h1c, src, dst2, s1f, asad1[:, 0], asad1[:, 1], zn)
    den1 = den1[:N].reshape(N, 1)
    h2c, asad2 = _tc_comb1(outp1, h1c, asad1, ssum1, ideg, den1,
                           b1.reshape(1, D), W2, a2)

    outp2, den2 = _sc_conv(h2c, src, dst2, s2f, asad2[:, 0], asad2[:, 1], zn)
    den2 = den2[:N].reshape(N, 1)
    hp1c, asadp1, hp2c, asadp2 = _tc_comb2(
        outp2, h2c, asad2, ssum2, ideg, den2, b2.reshape(1, D),
        Wp1, ap1, Wp2, ap2)

    outpp1, denp1 = _sc_conv(hp1c, src, dst2, zedge, asadp1[:, 0], asadp1[:, 1], zn)
    outpp2, denp2 = _sc_conv(hp2c, src, dst2, zedge, asadp2[:, 0], asadp2[:, 1], zn)
    denp1 = denp1[:N].reshape(N, 1)
    denp2 = denp2[:N].reshape(N, 1)

    return _tc_final(
        outpp1, hp1c, asadp1, denp1, bp1.reshape(1, D), wj1, bj1,
        outpp2, hp2c, asadp2, denp2, bp2.reshape(1, D), wj2, bj2, batchr)


# speedup vs baseline: 9.9643x; 1.4115x over previous
import functools

import jax
import jax.numpy as jnp
from jax import lax
from jax.experimental import pallas as pl
from jax.experimental.pallas import tpu as pltpu
from jax.experimental.pallas import tpu_sc as plsc

F32 = jnp.float32
I32 = jnp.int32

N = 10000
E = 160000
D = 512
C = 8
CW = 64
G = 16
NC = 2
NS = 16
L = 16
NPAD = 10240
EP = E // NS
GE = 80
NG = EP // GE
EPS = E // (NC * NS)
SP = N // NS
RED = NPAD // NS
BLKN = 1000
BLKE = 2000


def _leaky(a):
    return jnp.where(a > 0, a, 0.2 * a)


def _tc_lin0_body(x_ref, wnp_ref, bnp_ref, w1_ref, a1_ref, hc_ref, asad_ref):
    h0 = jnp.dot(x_ref[...], wnp_ref[...], preferred_element_type=F32) + bnp_ref[...]
    h1 = jnp.dot(h0, w1_ref[...], preferred_element_type=F32)
    for c in range(C):
        hc_ref[c] = h1[:, c * CW:(c + 1) * CW]
    asad_ref[...] = jnp.dot(h1, a1_ref[...], preferred_element_type=F32)


def _tc_lin0(x, wnp, bnp, w1, a1):
    nb = N // BLKN
    return pl.pallas_call(
        _tc_lin0_body,
        grid=(nb,),
        in_specs=[
            pl.BlockSpec((BLKN, 256), lambda i: (i, 0)),
            pl.BlockSpec((256, 256), lambda i: (0, 0)),
            pl.BlockSpec((1, 256), lambda i: (0, 0)),
            pl.BlockSpec((256, D), lambda i: (0, 0)),
            pl.BlockSpec((D, 8), lambda i: (0, 0)),
        ],
        out_specs=[
            pl.BlockSpec((C, BLKN, CW), lambda i: (0, i, 0)),
            pl.BlockSpec((BLKN, 8), lambda i: (i, 0)),
        ],
        out_shape=[
            jax.ShapeDtypeStruct((C, N, CW), F32),
            jax.ShapeDtypeStruct((N, 8), F32),
        ],
    )(x, wnp, bnp, w1, a1)


def _tc_es_body(ea_ref, u8_ref, c8_ref, es_ref):
    es_ref[...] = jnp.dot(ea_ref[...], u8_ref[...], preferred_element_type=F32) + c8_ref[...]


def _tc_es(ea, u8, c8):
    nb = E // BLKE
    return pl.pallas_call(
        _tc_es_body,
        grid=(nb,),
        in_specs=[
            pl.BlockSpec((BLKE, 256), lambda i: (i, 0)),
            pl.BlockSpec((256, 8), lambda i: (0, 0)),
            pl.BlockSpec((1, 8), lambda i: (0, 0)),
        ],
        out_specs=pl.BlockSpec((BLKE, 8), lambda i: (i, 0)),
        out_shape=jax.ShapeDtypeStruct((E, 8), F32),
    )(ea, u8, c8)


def _combine_block(outp_ref, hcin_ref, asad_ref, den_ref, b_ref, hsc, exl_extra, relu):
    a_s = asad_ref[:, 0:1]
    a_d = asad_ref[:, 1:2]
    exl = jnp.exp(_leaky(a_s + a_d + exl_extra))
    dt = den_ref[...] + exl + 1e-16
    for c in range(C):
        ob = (outp_ref[c] + exl * hcin_ref[c]) / dt + b_ref[:, c * CW:(c + 1) * CW]
        if relu:
            ob = jnp.maximum(ob, 0.0)
        hsc[:, c * CW:(c + 1) * CW] = ob


def _tc_comb1_body(outp_ref, hcin_ref, asad_ref, ssum_ref, ideg_ref, den_ref,
                   b_ref, w2_ref, a2_ref, hc2_ref, asad2_ref, hsc):
    m = ssum_ref[...] / jnp.maximum(ideg_ref[...], 1.0)
    _combine_block(outp_ref, hcin_ref, asad_ref, den_ref, b_ref, hsc, m, True)
    h = hsc[...]
    h2 = jnp.dot(h, w2_ref[...], preferred_element_type=F32)
    for c in range(C):
        hc2_ref[c] = h2[:, c * CW:(c + 1) * CW]
    asad2_ref[...] = jnp.dot(h2, a2_ref[...], preferred_element_type=F32)


def _tc_comb1(outp, hcin, asad, ssum, ideg, den, b, w2, a2):
    nb = N // BLKN
    col = pl.BlockSpec((BLKN, 1), lambda i: (i, 0))
    return pl.pallas_call(
        _tc_comb1_body,
        grid=(nb,),
        in_specs=[
            pl.BlockSpec((C, BLKN, CW), lambda i: (0, i, 0)),
            pl.BlockSpec((C, BLKN, CW), lambda i: (0, i, 0)),
            pl.BlockSpec((BLKN, 8), lambda i: (i, 0)),
            col, col, col,
            pl.BlockSpec((1, D), lambda i: (0, 0)),
            pl.BlockSpec((D, D), lambda i: (0, 0)),
            pl.BlockSpec((D, 8), lambda i: (0, 0)),
        ],
        out_specs=[
            pl.BlockSpec((C, BLKN, CW), lambda i: (0, i, 0)),
            pl.BlockSpec((BLKN, 8), lambda i: (i, 0)),
        ],
        out_shape=[
            jax.ShapeDtypeStruct((C, N, CW), F32),
            jax.ShapeDtypeStruct((N, 8), F32),
        ],
        scratch_shapes=[pltpu.VMEM((BLKN, D), F32)],
    )(outp, hcin, asad, ssum, ideg, den, b, w2, a2)


def _tc_comb2_body(outp_ref, hcin_ref, asad_ref, ssum_ref, ideg_ref, den_ref,
                   b_ref, wp1_ref, ap1_ref, wp2_ref, ap2_ref,
                   hp1_ref, asadp1_ref, hp2_ref, asadp2_ref, hsc):
    m = ssum_ref[...] / jnp.maximum(ideg_ref[...], 1.0)
    _combine_block(outp_ref, hcin_ref, asad_ref, den_ref, b_ref, hsc, m, True)
    h = hsc[...]
    hp1 = jnp.dot(h, wp1_ref[...], preferred_element_type=F32)
    for c in range(C):
        hp1_ref[c] = hp1[:, c * CW:(c + 1) * CW]
    asadp1_ref[...] = jnp.dot(hp1, ap1_ref[...], preferred_element_type=F32)
    hp2 = jnp.dot(h, wp2_ref[...], preferred_element_type=F32)
    for c in range(C):
        hp2_ref[c] = hp2[:, c * CW:(c + 1) * CW]
    asadp2_ref[...] = jnp.dot(hp2, ap2_ref[...], preferred_element_type=F32)


def _tc_comb2(outp, hcin, asad, ssum, ideg, den, b, wp1, ap1, wp2, ap2):
    nb = N // BLKN
    col = pl.BlockSpec((BLKN, 1), lambda i: (i, 0))
    return pl.pallas_call(
        _tc_comb2_body,
        grid=(nb,),
        in_specs=[
            pl.BlockSpec((C, BLKN, CW), lambda i: (0, i, 0)),
            pl.BlockSpec((C, BLKN, CW), lambda i: (0, i, 0)),
            pl.BlockSpec((BLKN, 8), lambda i: (i, 0)),
            col, col, col,
            pl.BlockSpec((1, D), lambda i: (0, 0)),
            pl.BlockSpec((D, D), lambda i: (0, 0)),
            pl.BlockSpec((D, 8), lambda i: (0, 0)),
            pl.BlockSpec((D, D), lambda i: (0, 0)),
            pl.BlockSpec((D, 8), lambda i: (0, 0)),
        ],
        out_specs=[
            pl.BlockSpec((C, BLKN, CW), lambda i: (0, i, 0)),
            pl.BlockSpec((BLKN, 8), lambda i: (i, 0)),
            pl.BlockSpec((C, BLKN, CW), lambda i: (0, i, 0)),
            pl.BlockSpec((BLKN, 8), lambda i: (i, 0)),
        ],
        out_shape=[
            jax.ShapeDtypeStruct((C, N, CW), F32),
            jax.ShapeDtypeStruct((N, 8), F32),
            jax.ShapeDtypeStruct((C, N, CW), F32),
            jax.ShapeDtypeStruct((N, 8), F32),
        ],
        scratch_shapes=[pltpu.VMEM((BLKN, D), F32)],
    )(outp, hcin, asad, ssum, ideg, den, b, wp1, ap1, wp2, ap2)


def _tc_final_body(outp1_ref, h1c_ref, as1_ref, den1_ref, b1_ref, wj1_ref, bj1_ref,
                   outp2_ref, h2c_ref, as2_ref, den2_ref, b2_ref, wj2_ref, bj2_ref,
                   batch_ref, out_ref, psc, g1sc, g2sc, cntsc):
    i = pl.program_id(0)

    @pl.when(i == 0)
    def _():
        g1sc[...] = jnp.zeros_like(g1sc)
        g2sc[...] = jnp.zeros_like(g2sc)
        cntsc[...] = jnp.zeros_like(cntsc)

    bat = batch_ref[...]
    oh = (lax.broadcasted_iota(I32, (BLKN, G), 1) == bat).astype(F32)
    ones = jnp.ones((BLKN, 1), F32)
    cntsc[:, 0:1] += lax.dot_general(oh, ones, (((0,), (0,)), ((), ())),
                                     preferred_element_type=F32)

    pools = [
        (outp1_ref, h1c_ref, as1_ref, den1_ref, b1_ref, wj1_ref, bj1_ref, g1sc),
        (outp2_ref, h2c_ref, as2_ref, den2_ref, b2_ref, wj2_ref, bj2_ref, g2sc),
    ]
    for (outp_ref, hc_ref, as_ref, den_ref, b_ref, wj_ref, bj_ref, gsc) in pools:
        _combine_block(outp_ref, hc_ref, as_ref, den_ref, b_ref, psc, 0.0, False)
        p = psc[...]
        sv = jnp.dot(p, wj_ref[...], preferred_element_type=F32)
        score = jax.nn.sigmoid(sv[:, 0:1] + bj_ref[:, 0:1])
        xw = score * p
        gsc[...] += lax.dot_general(oh, xw, (((0,), (0,)), ((), ())),
                                    preferred_element_type=F32)

    @pl.when(i == pl.num_programs(0) - 1)
    def _():
        cnt = jnp.maximum(cntsc[:, 0:1], 1.0)
        out_ref[:, :D] = g1sc[...] / cnt
        out_ref[:, D:] = g2sc[...] / cnt


def _tc_final(outp1, h1c, as1, den1, b1, wj1, bj1,
              outp2, h2c, as2, den2, b2, wj2, bj2, batchr):
    nb = N // BLKN
    col = pl.BlockSpec((BLKN, 1), lambda i: (i, 0))
    chunks = pl.BlockSpec((C, BLKN, CW), lambda i: (0, i, 0))
    asads = pl.BlockSpec((BLKN, 8), lambda i: (i, 0))
    full_b = pl.BlockSpec((1, D), lambda i: (0, 0))
    wj_s = pl.BlockSpec((D, 8), lambda i: (0, 0))
    bj_s = pl.BlockSpec((1, 8), lambda i: (0, 0))
    return pl.pallas_call(
        _tc_final_body,
        grid=(nb,),
        in_specs=[
            chunks, chunks, asads, col, full_b, wj_s, bj_s,
            chunks, chunks, asads, col, full_b, wj_s, bj_s,
            pl.BlockSpec((BLKN, 1), lambda i: (i, 0)),
        ],
        out_specs=pl.BlockSpec((G, 2 * D), lambda i: (0, 0)),
        out_shape=jax.ShapeDtypeStruct((G, 2 * D), F32),
        scratch_shapes=[
            pltpu.VMEM((BLKN, D), F32),
            pltpu.VMEM((G, D), F32),
            pltpu.VMEM((G, D), F32),
            pltpu.VMEM((G, CW), F32),
        ],
    )(outp1, h1c, as1, den1, b1, wj1, bj1,
      outp2, h2c, as2, den2, b2, wj2, bj2, batchr)


_MESH = dict(core_axis_name="c", subcore_axis_name="s", num_cores=NC,
             num_subcores=NS)


def _sc_conv_body(hc, srcf, dst2, sflat, asf, adf, zn,
                  outp, den_out,
                  a_s_tbl, a_d_tbl, srcb, dstb, sb, exb, dtbl,
                  rows0, rows1, rtmp, gsem0, gsem1, ssem0, ssem1, acc, spred):
    cid = lax.axis_index("c")
    sid = lax.axis_index("s")
    z16 = jnp.zeros((L,), F32)

    pltpu.sync_copy(asf, a_s_tbl.at[pl.ds(0, N)])
    pltpu.sync_copy(adf, a_d_tbl.at[pl.ds(0, N)])
    e0 = pl.multiple_of(sid * EP, 8)
    pltpu.sync_copy(srcf.at[pl.ds(e0, EP)], srcb)
    pltpu.sync_copy(dst2.at[sid], dstb)
    pltpu.sync_copy(sflat.at[pl.ds(e0, EP)], sb)

    def zd(v, _):
        dtbl[pl.ds(v * L, L)] = z16
        return _
    lax.fori_loop(0, NPAD // L, zd, None)

    def chunk_pass(ci, _carry):
        cc = cid * (C // NC) + ci
        r0 = pl.multiple_of(sid * 632, 8)

        @pl.when(sid < NS - 1)
        def _():
            sl = pl.ds(r0, 632)
            pltpu.sync_copy(zn.at[sl], acc.at[sl])

        @pl.when(sid == NS - 1)
        def _():
            sl = pl.ds(9480, 520)
            pltpu.sync_copy(zn.at[sl], acc.at[sl])
        plsc.subcore_barrier()

        hcc = hc.at[cc]

        def compute_ex(g):
            base = g * GE
            for j in range(GE // L):
                off = base + j * L
                si = srcb[pl.ds(off, L)]
                di = dstb[g, pl.ds(j * L, L)]
                av = (plsc.load_gather(a_s_tbl, [si])
                      + plsc.load_gather(a_d_tbl, [di])
                      + sb[pl.ds(off, L)])
                ex = jnp.exp(_leaky(av))
                exb[pl.ds(off, L)] = ex

                @pl.when(cid == 0)
                def _():
                    plsc.addupdate_scatter(dtbl, [di], ex)

        def scale(rbuf, g):
            base = g * GE

            def sj(j, _):
                exv = plsc.load_gather(exb, [jnp.full((L,), base + j, I32)])
                for k in range(CW // L):
                    sl = pl.ds(k * L, L)
                    rbuf[j, sl] = rbuf[j, sl] * exv
                return _
            lax.fori_loop(0, GE, sj, None, unroll=4)

        def gather_src(g, rbuf):
            return hcc.at[srcb.at[pl.ds(g * GE, GE)]], rbuf

        pltpu.async_copy(*gather_src(0, rows0), gsem0)

        def pair(tt, _):
            g0 = 2 * tt
            g1 = g0 + 1

            @pl.when(tt > 0)
            def _():
                pltpu.make_async_copy(rows1, acc.at[dstb.at[g0 - 1]], ssem1).wait()
            pltpu.async_copy(*gather_src(g1, rows1), gsem1)
            pltpu.make_async_copy(*gather_src(g0, rows0), gsem0).wait()

            @pl.when(ci == 0)
            def _():
                compute_ex(g0)
            scale(rows0, g0)
            pltpu.async_copy(rows0, acc.at[dstb.at[g0]], ssem0, add=True)
            pltpu.make_async_copy(*gather_src(g1, rows1), gsem1).wait()

            @pl.when(ci == 0)
            def _():
                compute_ex(g1)
            scale(rows1, g1)
            pltpu.make_async_copy(rows0, acc.at[dstb.at[g0]], ssem0).wait()

            @pl.when(g0 + 2 < NG)
            def _():
                pltpu.async_copy(*gather_src(g0 + 2, rows0), gsem0)
            pltpu.async_copy(rows1, acc.at[dstb.at[g1]], ssem1, add=True)
            return _

        lax.fori_loop(0, NG // 2, pair, None)
        gl = NG - 1
        pltpu.make_async_copy(*gather_src(gl, rows0), gsem0).wait()

        @pl.when(ci == 0)
        def _():
            compute_ex(gl)
        scale(rows0, gl)
        pltpu.make_async_copy(rows1, acc.at[dstb.at[gl - 1]], ssem1).wait()
        pltpu.sync_copy(rows0, acc.at[dstb.at[gl]], add=True)
        plsc.subcore_barrier()

        @pl.when(sid < NS - 1)
        def _():
            sl = pl.ds(r0, 632)
            pltpu.sync_copy(acc.at[sl], outp.at[cc].at[sl])

        @pl.when(sid == NS - 1)
        def _():
            sl = pl.ds(9480, 520)
            pltpu.sync_copy(acc.at[sl], outp.at[cc].at[sl])
        plsc.subcore_barrier()
        return _carry

    lax.fori_loop(0, C // NC, chunk_pass, None)

    @pl.when(cid == 0)
    def _():
        for r in range(NPAD // 2560):
            rb = r * 2560
            s0 = pl.multiple_of(sid * 160, 8)
            n0 = pl.multiple_of(rb + sid * 160, 8)
            pltpu.sync_copy(dtbl.at[pl.ds(rb, 2560)], spred.at[sid])
            plsc.subcore_barrier()

            def zr(v, _):
                dtbl[pl.ds(n0 + v * L, L)] = z16
                return _
            lax.fori_loop(0, 160 // L, zr, None)
            for tt in range(NS):
                pltpu.sync_copy(spred.at[tt].at[pl.ds(s0, 160)],
                                rtmp.at[pl.ds(0, 160)])

                def addv(v, _):
                    sl = pl.ds(n0 + v * L, L)
                    dtbl[sl] = dtbl[sl] + rtmp[pl.ds(v * L, L)]
                    return _
                lax.fori_loop(0, 160 // L, addv, None)
            pltpu.sync_copy(dtbl.at[pl.ds(n0, 160)], den_out.at[pl.ds(n0, 160)])
            plsc.subcore_barrier()


def _sc_conv(hc, srcf, dst2, sflat, asf, adf, zn):
    mesh = plsc.VectorSubcoreMesh(**_MESH)
    fn = pl.kernel(
        _sc_conv_body,
        out_type=[
            jax.ShapeDtypeStruct((C, N, CW), F32),
            jax.ShapeDtypeStruct((NPAD,), F32),
        ],
        mesh=mesh,
        compiler_params=pltpu.CompilerParams(needs_layout_passes=False,
                                             use_tc_tiling_on_sc=False),
        scratch_types=[
            pltpu.VMEM((NPAD,), F32),
            pltpu.VMEM((NPAD,), F32),
            pltpu.VMEM((EP,), I32),
            pltpu.VMEM((NG, GE), I32),
            pltpu.VMEM((EP,), F32),
            pltpu.VMEM((EP,), F32),
            pltpu.VMEM((NPAD,), F32),
            pltpu.VMEM((GE, CW), F32),
            pltpu.VMEM((GE, CW), F32),
            pltpu.VMEM((RED,), F32),
            pltpu.SemaphoreType.DMA,
            pltpu.SemaphoreType.DMA,
            pltpu.SemaphoreType.DMA,
            pltpu.SemaphoreType.DMA,
            pltpu.VMEM_SHARED((N, CW), F32),
            pltpu.VMEM_SHARED((NS, 2560), F32),
        ],
    )
    return fn(hc, srcf, dst2, sflat, asf, adf, zn)


def _sc_stats_body(dstf, s1f, s2f, s1o, s2o, cnto,
                   dstb, sb1, sb2, t1, t2, tc, redb, rtmp, spred):
    cid = lax.axis_index("c")
    sid = lax.axis_index("s")
    wid = cid * NS + sid
    e0 = pl.multiple_of(wid * EPS, 8)
    iota = lax.iota(I32, L)
    z16 = jnp.zeros((L,), F32)
    ones = jnp.ones((L,), F32)

    pltpu.sync_copy(dstf.at[pl.ds(e0, EPS)], dstb.at[pl.ds(0, EPS)])
    pltpu.sync_copy(s1f.at[pl.ds(e0, EPS)], sb1.at[pl.ds(0, EPS)])
    pltpu.sync_copy(s2f.at[pl.ds(e0, EPS)], sb2.at[pl.ds(0, EPS)])

    for t in (t1, t2, tc):
        def zd(v, _, _t=t):
            _t[pl.ds(v * L, L)] = z16
            return _
        lax.fori_loop(0, NPAD // L, zd, None)

    def grp(g, _):
        off = g * L
        msk = (off + iota) < EPS
        di = dstb[pl.ds(off, L)]
        plsc.addupdate_scatter(t1, [di], sb1[pl.ds(off, L)], mask=msk)
        plsc.addupdate_scatter(t2, [di], sb2[pl.ds(off, L)], mask=msk)
        plsc.addupdate_scatter(tc, [di], ones, mask=msk)
        return _
    lax.fori_loop(0, (EPS + L - 1) // L, grp, None)

    n0 = pl.multiple_of(sid * RED, 8)
    for (tbl, outr) in ((t1, s1o), (t2, s2o), (tc, cnto)):
        plsc.subcore_barrier()
        pltpu.sync_copy(tbl, spred.at[sid])
        plsc.subcore_barrier()

        def zr(v, _):
            redb[pl.ds(v * L, L)] = z16
            return _
        lax.fori_loop(0, RED // L, zr, None)
        for t in range(NS):
            pltpu.sync_copy(spred.at[t].at[pl.ds(n0, RED)], rtmp)

            def addv(v, _):
                sl = pl.ds(v * L, L)
                redb[sl] = redb[sl] + rtmp[sl]
                return _
            lax.fori_loop(0, RED // L, addv, None)
        pltpu.sync_copy(redb, outr.at[cid].at[pl.ds(n0, RED)])


def _sc_stats(dstf, s1f, s2f):
    mesh = plsc.VectorSubcoreMesh(**_MESH)
    fn = pl.kernel(
        _sc_stats_body,
        out_type=[
            jax.ShapeDtypeStruct((NC, NPAD), F32),
            jax.ShapeDtypeStruct((NC, NPAD), F32),
            jax.ShapeDtypeStruct((NC, NPAD), F32),
        ],
        mesh=mesh,
        compiler_params=pltpu.CompilerParams(needs_layout_passes=False,
                                             use_tc_tiling_on_sc=False),
        scratch_types=[
            pltpu.VMEM((EPS + 16, ), I32),
            pltpu.VMEM((EPS + 16, ), F32),
            pltpu.VMEM((EPS + 16, ), F32),
            pltpu.VMEM((NPAD,), F32),
            pltpu.VMEM((NPAD,), F32),
            pltpu.VMEM((NPAD,), F32),
            pltpu.VMEM((RED,), F32),
            pltpu.VMEM((RED,), F32),
            pltpu.VMEM_SHARED((NS, NPAD), F32),
        ],
    )
    return fn(dstf, s1f, s2f)


def kernel(x, edge_index, edge_attr, batch, Wnp, bnp, Wep, bep, W1, as1, ad1, b1,
           We1, ae1, W2, as2, ad2, b2, We2, ae2, Wp1, asp1, adp1, bp1, wproj1,
           bproj1, Wp2, asp2, adp2, bp2, wproj2, bproj2):
    src = edge_index[0]
    dst = edge_index[1]
    dst2 = dst.reshape(NS, NG, GE)

    v1 = We1 @ ae1
    u1 = Wep @ v1
    c1 = bep @ v1
    v2 = We2 @ ae2
    u2 = Wep @ v2
    c2 = bep @ v2
    u8 = jnp.zeros((256, 8), F32).at[:, 0].set(u1).at[:, 1].set(u2)
    c8 = jnp.zeros((1, 8), F32).at[0, 0].set(c1).at[0, 1].set(c2)

    def cols8(a, b):
        return jnp.zeros((D, 8), F32).at[:, 0].set(a).at[:, 1].set(b)

    a1 = cols8(as1, ad1)
    a2 = cols8(as2, ad2)
    ap1 = cols8(asp1, adp1)
    ap2 = cols8(asp2, adp2)
    wj1 = jnp.zeros((D, 8), F32).at[:, 0].set(wproj1[:, 0])
    wj2 = jnp.zeros((D, 8), F32).at[:, 0].set(wproj2[:, 0])
    bj1 = jnp.zeros((1, 8), F32).at[0, 0].set(bproj1[0])
    bj2 = jnp.zeros((1, 8), F32).at[0, 0].set(bproj2[0])
    zn = jnp.zeros((N, CW), F32)
    batchr = batch.reshape(N, 1)

    h1c, asad1 = _tc_lin0(x, Wnp, bnp.reshape(1, 256), W1, a1)
    es = _tc_es(edge_attr, u8, c8)

    s1f = es[:, 0]
    s2f = es[:, 1]
    zedge = jnp.zeros((E,), F32)
    s1p, s2p, cntp = _sc_stats(dst, s1f, s2f)
    ssum1 = (s1p[0] + s1p[1])[:N].reshape(N, 1)
    ssum2 = (s2p[0] + s2p[1])[:N].reshape(N, 1)
    ideg = (cntp[0] + cntp[1])[:N].reshape(N, 1)

    outp1, den1 = _sc_conv(h1c, src, dst2, s1f, asad1[:, 0], asad1[:, 1], zn)
    den1 = den1[:N].reshape(N, 1)
    h2c, asad2 = _tc_comb1(outp1, h1c, asad1, ssum1, ideg, den1,
                           b1.reshape(1, D), W2, a2)

    outp2, den2 = _sc_conv(h2c, src, dst2, s2f, asad2[:, 0], asad2[:, 1], zn)
    den2 = den2[:N].reshape(N, 1)
    hp1c, asadp1, hp2c, asadp2 = _tc_comb2(
        outp2, h2c, asad2, ssum2, ideg, den2, b2.reshape(1, D),
        Wp1, ap1, Wp2, ap2)

    outpp1, denp1 = _sc_conv(hp1c, src, dst2, zedge, asadp1[:, 0], asadp1[:, 1], zn)
    outpp2, denp2 = _sc_conv(hp2c, src, dst2, zedge, asadp2[:, 0], asadp2[:, 1], zn)
    denp1 = denp1[:N].reshape(N, 1)
    denp2 = denp2[:N].reshape(N, 1)

    return _tc_final(
        outpp1, hp1c, asadp1, denp1, bp1.reshape(1, D), wj1, bj1,
        outpp2, hp2c, asadp2, denp2, bp2.reshape(1, D), wj2, bj2, batchr)
